# baseline (speedup 1.0000x reference)
import functools

import jax
import jax.numpy as jnp
import numpy as np
from jax import lax
from jax.experimental import pallas as pl
from jax.experimental.pallas import tpu as pltpu
from jax.experimental.pallas import tpu_sc as plsc

B = 128
V = 100000
W = 1024
NT = V // W
TW = V - NT * W
BR = 8
NR = B // BR
UNROLL = 4
UNROLL2 = 4

_K1 = np.uint32(0)
_K2 = np.uint32(42)
_TINY = np.float32(1.1754943508222875e-38)
_I32_BIG = np.int32(2**31 - 1)


def _threefry_bits(cnt_lo):
    ks0 = _K1
    ks1 = _K2
    ks2 = np.uint32(_K1 ^ _K2 ^ np.uint32(0x1BD11BDA))

    x0 = jnp.zeros_like(cnt_lo) + ks0
    x1 = cnt_lo + ks1

    def rnd(x0, x1, r):
        x0 = x0 + x1
        x1 = (x1 << np.uint32(r)) | (x1 >> np.uint32(32 - r))
        x1 = x0 ^ x1
        return x0, x1

    for r in (13, 15, 26, 6):
        x0, x1 = rnd(x0, x1, r)
    x0 = x0 + ks1
    x1 = x1 + np.uint32(ks2 + np.uint32(1))
    for r in (17, 29, 16, 24):
        x0, x1 = rnd(x0, x1, r)
    x0 = x0 + ks2
    x1 = x1 + np.uint32(ks0 + np.uint32(2))
    for r in (13, 15, 26, 6):
        x0, x1 = rnd(x0, x1, r)
    x0 = x0 + ks0
    x1 = x1 + np.uint32(ks1 + np.uint32(3))
    for r in (17, 29, 16, 24):
        x0, x1 = rnd(x0, x1, r)
    x0 = x0 + ks1
    x1 = x1 + np.uint32(ks2 + np.uint32(4))
    for r in (13, 15, 26, 6):
        x0, x1 = rnd(x0, x1, r)
    x0 = x0 + ks2
    x1 = x1 + np.uint32(ks0 + np.uint32(5))
    return x0 ^ x1


def _gumbel_score(blk, cnt):
    bits = _threefry_bits(cnt.astype(jnp.uint32))
    fbits = (bits >> np.uint32(9)) | np.uint32(0x3F800000)
    floats = lax.bitcast_convert_type(fbits, jnp.float32) - np.float32(1.0)
    u = jnp.maximum(_TINY, floats + _TINY)
    return blk - jnp.log(-jnp.log(u))


def _dense_body(logits_ref, lp_ref, idx_ref):
    j = pl.program_id(0)
    row0 = j * BR

    rowi = lax.broadcasted_iota(jnp.int32, (BR, W), 0)
    lane = lax.broadcasted_iota(jnp.int32, (BR, W), 1)
    cnt0 = (row0 + rowi) * V + lane

    def chunk(t, carry):
        a_score, a_cnt, a_lmax, a_se = carry
        c0 = pl.multiple_of(t * W, W)
        blk = logits_ref[:, pl.ds(c0, W)]
        cnt = cnt0 + t * W
        score = _gumbel_score(blk, cnt)
        upd = score > a_score
        a_score = jnp.where(upd, score, a_score)
        a_cnt = jnp.where(upd, cnt, a_cnt)
        nm = jnp.maximum(a_lmax, blk)
        a_se = a_se * jnp.exp(a_lmax - nm) + jnp.exp(blk - nm)
        return a_score, a_cnt, nm, a_se

    neg_inf = jnp.full((BR, W), -jnp.inf, jnp.float32)
    a_score, a_cnt, a_lmax, se = lax.fori_loop(
        0, NT, chunk,
        (neg_inf, jnp.zeros((BR, W), jnp.int32), neg_inf,
         jnp.zeros((BR, W), jnp.float32)),
        unroll=UNROLL)

    tblk = logits_ref[:, NT * W:V]
    tcnt = ((row0 + lax.broadcasted_iota(jnp.int32, (BR, TW), 0)) * V
            + NT * W + lax.broadcasted_iota(jnp.int32, (BR, TW), 1))
    tscore = _gumbel_score(tblk, tcnt)

    m_main = jnp.max(a_score, axis=1)
    m_tail = jnp.max(tscore, axis=1)
    m_sc = jnp.maximum(m_main, m_tail)
    cand_m = jnp.min(jnp.where(a_score == m_sc[:, None], a_cnt, _I32_BIG),
                     axis=1)
    cand_t = jnp.min(jnp.where(tscore == m_sc[:, None], tcnt, _I32_BIG),
                     axis=1)
    cnt_win = jnp.minimum(cand_m, cand_t)

    lm_t = jnp.max(tblk, axis=1)
    m_row = jnp.maximum(jnp.max(a_lmax, axis=1), lm_t)
    se_row = (jnp.sum(se * jnp.exp(a_lmax - m_row[:, None]), axis=1)
              + jnp.sum(jnp.exp(tblk - m_row[:, None]), axis=1))

    bits_w = _threefry_bits(cnt_win.astype(jnp.uint32))
    fb_w = (bits_w >> np.uint32(9)) | np.uint32(0x3F800000)
    fl_w = lax.bitcast_convert_type(fb_w, jnp.float32) - np.float32(1.0)
    u_w = jnp.maximum(_TINY, fl_w + _TINY)
    l_win = m_sc + jnp.log(-jnp.log(u_w))

    rvec = cnt0[:, 0]
    lp_ref[j, :] = l_win - (m_row + jnp.log(se_row))
    idx_ref[j, :] = cnt_win - rvec


def _dense_call(logits, interpret=False):
    lp2, idx2 = pl.pallas_call(
        _dense_body,
        grid=(NR,),
        in_specs=[pl.BlockSpec((BR, V), lambda j: (j, 0))],
        out_specs=[pl.BlockSpec((NR, BR), lambda j: (0, 0)),
                   pl.BlockSpec((NR, BR), lambda j: (0, 0))],
        out_shape=[jax.ShapeDtypeStruct((NR, BR), jnp.float32),
                   jax.ShapeDtypeStruct((NR, BR), jnp.int32)],
        compiler_params=pltpu.CompilerParams(
            dimension_semantics=("parallel",)),
        interpret=interpret,
    )(logits)
    return lp2.reshape(B), idx2.reshape(B)


def _sc_gather(action_space, idx):
    info = plsc.get_sparse_core_info()
    nc = info.num_cores
    n_workers = 16
    per_w = B // n_workers
    mesh = plsc.VectorSubcoreMesh(core_axis_name="c", subcore_axis_name="s")

    @functools.partial(
        pl.kernel, mesh=mesh,
        out_type=jax.ShapeDtypeStruct((B,), jnp.int32),
        scratch_types=[pltpu.VMEM((per_w,), jnp.int32),
                       pltpu.VMEM((per_w,), jnp.int32),
                       pltpu.SemaphoreType.DMA],
    )
    def gather_kernel(table_hbm, idx_hbm, out_hbm, idx_v, rows_v, sem):
        wid = lax.axis_index("s") * nc + lax.axis_index("c")

        @pl.when(wid < n_workers)
        def _():
            base = wid * per_w
            pltpu.sync_copy(idx_hbm.at[pl.ds(base, per_w)], idx_v)
            pltpu.async_copy(table_hbm.at[idx_v], rows_v, sem).wait()
            pltpu.sync_copy(rows_v, out_hbm.at[pl.ds(base, per_w)])

    return gather_kernel(action_space, idx)


def kernel(logits, action_space):
    log_probs, idx = _dense_call(logits)
    actions = _sc_gather(action_space, idx)
    return log_probs, actions


# speedup vs baseline: 1.0202x; 1.0202x over previous
import functools

import jax
import jax.numpy as jnp
import numpy as np
from jax import lax
from jax.experimental import pallas as pl
from jax.experimental.pallas import tpu as pltpu
from jax.experimental.pallas import tpu_sc as plsc

B = 128
V = 100000
W = 1024
NT = V // W
TW = V - NT * W
BR = 8
NR = B // BR
UNROLL = 8
UNROLL2 = 8

_K1 = np.uint32(0)
_K2 = np.uint32(42)
_TINY = np.float32(1.1754943508222875e-38)
_I32_BIG = np.int32(2**31 - 1)


def _threefry_bits(cnt_lo):
    ks0 = _K1
    ks1 = _K2
    ks2 = np.uint32(_K1 ^ _K2 ^ np.uint32(0x1BD11BDA))

    x0 = jnp.zeros_like(cnt_lo) + ks0
    x1 = cnt_lo + ks1

    def rnd(x0, x1, r):
        x0 = x0 + x1
        x1 = (x1 << np.uint32(r)) | (x1 >> np.uint32(32 - r))
        x1 = x0 ^ x1
        return x0, x1

    for r in (13, 15, 26, 6):
        x0, x1 = rnd(x0, x1, r)
    x0 = x0 + ks1
    x1 = x1 + np.uint32(ks2 + np.uint32(1))
    for r in (17, 29, 16, 24):
        x0, x1 = rnd(x0, x1, r)
    x0 = x0 + ks2
    x1 = x1 + np.uint32(ks0 + np.uint32(2))
    for r in (13, 15, 26, 6):
        x0, x1 = rnd(x0, x1, r)
    x0 = x0 + ks0
    x1 = x1 + np.uint32(ks1 + np.uint32(3))
    for r in (17, 29, 16, 24):
        x0, x1 = rnd(x0, x1, r)
    x0 = x0 + ks1
    x1 = x1 + np.uint32(ks2 + np.uint32(4))
    for r in (13, 15, 26, 6):
        x0, x1 = rnd(x0, x1, r)
    x0 = x0 + ks2
    x1 = x1 + np.uint32(ks0 + np.uint32(5))
    return x0 ^ x1


def _gumbel_score(blk, cnt):
    bits = _threefry_bits(cnt.astype(jnp.uint32))
    fbits = (bits >> np.uint32(9)) | np.uint32(0x3F800000)
    floats = lax.bitcast_convert_type(fbits, jnp.float32) - np.float32(1.0)
    u = jnp.maximum(_TINY, floats + _TINY)
    return blk - jnp.log(-jnp.log(u))


def _dense_body(logits_ref, lp_ref, idx_ref):
    j = pl.program_id(0)
    row0 = j * BR

    rowi = lax.broadcasted_iota(jnp.int32, (BR, W), 0)
    lane = lax.broadcasted_iota(jnp.int32, (BR, W), 1)
    cnt0 = (row0 + rowi) * V + lane

    def chunk(t, carry):
        a_score, a_cnt = carry
        c0 = pl.multiple_of(t * W, W)
        blk = logits_ref[:, pl.ds(c0, W)]
        cnt = cnt0 + t * W
        score = _gumbel_score(blk, cnt)
        upd = score > a_score
        return jnp.where(upd, score, a_score), jnp.where(upd, cnt, a_cnt)

    neg_inf = jnp.full((BR, W), -jnp.inf, jnp.float32)
    a_score, a_cnt = lax.fori_loop(
        0, NT, chunk, (neg_inf, jnp.zeros((BR, W), jnp.int32)),
        unroll=UNROLL)

    def chunk2a(t, a_lmax):
        c0 = pl.multiple_of(t * W, W)
        return jnp.maximum(a_lmax, logits_ref[:, pl.ds(c0, W)])

    a_lmax = lax.fori_loop(0, NT, chunk2a, neg_inf, unroll=UNROLL2)

    def chunk2b(t, se):
        c0 = pl.multiple_of(t * W, W)
        return se + jnp.exp(logits_ref[:, pl.ds(c0, W)] - a_lmax)

    se = lax.fori_loop(0, NT, chunk2b, jnp.zeros((BR, W), jnp.float32),
                       unroll=UNROLL2)

    tblk = logits_ref[:, NT * W:V]
    tcnt = ((row0 + lax.broadcasted_iota(jnp.int32, (BR, TW), 0)) * V
            + NT * W + lax.broadcasted_iota(jnp.int32, (BR, TW), 1))
    tscore = _gumbel_score(tblk, tcnt)

    m_main = jnp.max(a_score, axis=1)
    m_tail = jnp.max(tscore, axis=1)
    m_sc = jnp.maximum(m_main, m_tail)
    cand_m = jnp.min(jnp.where(a_score == m_sc[:, None], a_cnt, _I32_BIG),
                     axis=1)
    cand_t = jnp.min(jnp.where(tscore == m_sc[:, None], tcnt, _I32_BIG),
                     axis=1)
    cnt_win = jnp.minimum(cand_m, cand_t)

    lm_t = jnp.max(tblk, axis=1)
    m_row = jnp.maximum(jnp.max(a_lmax, axis=1), lm_t)
    se_row = (jnp.sum(se * jnp.exp(a_lmax - m_row[:, None]), axis=1)
              + jnp.sum(jnp.exp(tblk - m_row[:, None]), axis=1))

    bits_w = _threefry_bits(cnt_win.astype(jnp.uint32))
    fb_w = (bits_w >> np.uint32(9)) | np.uint32(0x3F800000)
    fl_w = lax.bitcast_convert_type(fb_w, jnp.float32) - np.float32(1.0)
    u_w = jnp.maximum(_TINY, fl_w + _TINY)
    l_win = m_sc + jnp.log(-jnp.log(u_w))

    rvec = cnt0[:, 0]
    lp_ref[j, :] = l_win - (m_row + jnp.log(se_row))
    idx_ref[j, :] = cnt_win - rvec


def _dense_call(logits, interpret=False):
    lp2, idx2 = pl.pallas_call(
        _dense_body,
        grid=(NR,),
        in_specs=[pl.BlockSpec((BR, V), lambda j: (j, 0))],
        out_specs=[pl.BlockSpec((NR, BR), lambda j: (0, 0)),
                   pl.BlockSpec((NR, BR), lambda j: (0, 0))],
        out_shape=[jax.ShapeDtypeStruct((NR, BR), jnp.float32),
                   jax.ShapeDtypeStruct((NR, BR), jnp.int32)],
        compiler_params=pltpu.CompilerParams(
            dimension_semantics=("arbitrary",)),
        interpret=interpret,
    )(logits)
    return lp2.reshape(B), idx2.reshape(B)


def _sc_gather(action_space, idx):
    info = plsc.get_sparse_core_info()
    nc = info.num_cores
    n_workers = 16
    per_w = B // n_workers
    mesh = plsc.VectorSubcoreMesh(core_axis_name="c", subcore_axis_name="s")

    @functools.partial(
        pl.kernel, mesh=mesh,
        out_type=jax.ShapeDtypeStruct((B,), jnp.int32),
        scratch_types=[pltpu.VMEM((per_w,), jnp.int32),
                       pltpu.VMEM((per_w,), jnp.int32),
                       pltpu.SemaphoreType.DMA],
    )
    def gather_kernel(table_hbm, idx_hbm, out_hbm, idx_v, rows_v, sem):
        wid = lax.axis_index("s") * nc + lax.axis_index("c")

        @pl.when(wid < n_workers)
        def _():
            base = wid * per_w
            pltpu.sync_copy(idx_hbm.at[pl.ds(base, per_w)], idx_v)
            pltpu.async_copy(table_hbm.at[idx_v], rows_v, sem).wait()
            pltpu.sync_copy(rows_v, out_hbm.at[pl.ds(base, per_w)])

    return gather_kernel(action_space, idx)


def kernel(logits, action_space):
    log_probs, idx = _dense_call(logits)
    actions = _sc_gather(action_space, idx)
    return log_probs, actions
